# Initial kernel scaffold; baseline (speedup 1.0000x reference)
import jax
import jax.numpy as jnp
from jax.experimental import pallas as pl


def kernel(x, edge_index, proj_W, proj_b, l1_Wl, l1_bl, l1_Wr, l2_Wl, l2_bl, l2_Wr):
    raise NotImplementedError("write your pallas kernel here")


# speedup vs baseline: 4.2098x; 4.2098x over previous
import functools

import jax
import jax.numpy as jnp
from jax import lax
from jax.experimental import pallas as pl
from jax.experimental.pallas import tpu as pltpu
from jax.experimental.pallas import tpu_sc as plsc

_NC = 2
_NS = 16
_IW = 128


def _tc1_body(x_ref, w_ref, b_ref, h0_ref, h1_ref, h2_ref, h3_ref):
    h = jnp.dot(x_ref[...], w_ref[...], preferred_element_type=jnp.float32)
    h = jnp.maximum(h + b_ref[...], 0.0)
    q = h.shape[1] // 4
    h0_ref[...] = h[:, 0 * q:1 * q]
    h1_ref[...] = h[:, 1 * q:2 * q]
    h2_ref[...] = h[:, 2 * q:3 * q]
    h3_ref[...] = h[:, 3 * q:4 * q]


def _tc1(x, wT, b):
    n, d = x.shape
    q = d // 4
    br = 2000
    return pl.pallas_call(
        _tc1_body,
        grid=(n // br,),
        in_specs=[
            pl.BlockSpec((br, d), lambda i: (i, 0)),
            pl.BlockSpec((d, d), lambda i: (0, 0)),
            pl.BlockSpec((1, d), lambda i: (0, 0)),
        ],
        out_specs=[pl.BlockSpec((br, q), lambda i: (i, 0))] * 4,
        out_shape=[jax.ShapeDtypeStruct((n, q), jnp.float32)] * 4,
    )(x, wT, b)


def _tc2_body(a0_ref, a1_ref, a2_ref, a3_ref, cnt_ref, x_ref,
              wl0_ref, wl1_ref, wl2_ref, wl3_ref, bl_ref,
              wr_ref, w2_ref, h1_ref, p_ref):
    inv = 1.0 / jnp.maximum(cnt_ref[...][:, :1], 1.0)
    aggmm = (jnp.dot(a0_ref[...], wl0_ref[...], preferred_element_type=jnp.float32)
             + jnp.dot(a1_ref[...], wl1_ref[...], preferred_element_type=jnp.float32)
             + jnp.dot(a2_ref[...], wl2_ref[...], preferred_element_type=jnp.float32)
             + jnp.dot(a3_ref[...], wl3_ref[...], preferred_element_type=jnp.float32))
    h1 = (inv * aggmm + bl_ref[...]
          + jnp.dot(x_ref[...], wr_ref[...], preferred_element_type=jnp.float32))
    h1_ref[...] = h1
    p_ref[...] = jnp.dot(h1, w2_ref[...], preferred_element_type=jnp.float32)


def _tc2(aggs, cnt8, x, wls, bl, wrT, w2):
    n, q = aggs[0].shape
    d = x.shape[1]
    h = wrT.shape[1]
    cp = w2.shape[1]
    br = 2000
    return pl.pallas_call(
        _tc2_body,
        grid=(n // br,),
        in_specs=(
            [pl.BlockSpec((br, q), lambda i: (i, 0))] * 4
            + [
                pl.BlockSpec((br, 8), lambda i: (i, 0)),
                pl.BlockSpec((br, d), lambda i: (i, 0)),
            ]
            + [pl.BlockSpec((q, h), lambda i: (0, 0))] * 4
            + [
                pl.BlockSpec((1, h), lambda i: (0, 0)),
                pl.BlockSpec((d, h), lambda i: (0, 0)),
                pl.BlockSpec((h, cp), lambda i: (0, 0)),
            ]
        ),
        out_specs=[
            pl.BlockSpec((br, h), lambda i: (i, 0)),
            pl.BlockSpec((br, cp), lambda i: (i, 0)),
        ],
        out_shape=[
            jax.ShapeDtypeStruct((n, h), jnp.float32),
            jax.ShapeDtypeStruct((n, cp), jnp.float32),
        ],
    )(*aggs, cnt8, x, *wls, bl, wrT, w2)


def _tc3_body(c_real, a2a_ref, a2b_ref, cnt_ref, h1_ref, wr2_ref, b2_ref, o_ref):
    inv = 1.0 / jnp.maximum(cnt_ref[...][:, :1], 1.0)
    logits = (inv * (a2a_ref[...] + a2b_ref[...]) + b2_ref[...]
              + jnp.dot(h1_ref[...], wr2_ref[...], preferred_element_type=jnp.float32))
    col = lax.broadcasted_iota(jnp.int32, logits.shape, 1)
    logits = jnp.where(col < c_real, logits, -1e30)
    m = jnp.max(logits, axis=1, keepdims=True)
    ls = jnp.log(jnp.sum(jnp.exp(logits - m), axis=1, keepdims=True))
    o_ref[...] = logits - m - ls


def _tc3(a2a, a2b, cnt8, h1, wr2, b2, c_real):
    n, cp = a2a.shape
    h = h1.shape[1]
    br = 2000
    return pl.pallas_call(
        functools.partial(_tc3_body, c_real),
        grid=(n // br,),
        in_specs=[
            pl.BlockSpec((br, cp), lambda i: (i, 0)),
            pl.BlockSpec((br, cp), lambda i: (i, 0)),
            pl.BlockSpec((br, 8), lambda i: (i, 0)),
            pl.BlockSpec((br, h), lambda i: (i, 0)),
            pl.BlockSpec((h, cp), lambda i: (0, 0)),
            pl.BlockSpec((1, cp), lambda i: (0, 0)),
        ],
        out_specs=pl.BlockSpec((br, cp), lambda i: (i, 0)),
        out_shape=jax.ShapeDtypeStruct((n, cp), jnp.float32),
    )(a2a, a2b, cnt8, h1, wr2, b2)


_CH = 40


def _chunk_loop(s, n, fn):
    nch = n // _CH
    bound = nch // _NS + jnp.where(s < (nch % _NS), 1, 0).astype(jnp.int32)

    def it(k, carry):
        fn((s + k * _NS) * _CH)
        return carry
    lax.fori_loop(0, bound, it, 0)


def _sc_agg_wide(hs, src2d, dst2d, zrow, zc8, ones8):
    n, q = hs[0].shape
    rows_total = src2d.shape[0]
    rpt = rows_total // _NS
    burst = 8
    npad = n + 8
    mesh = plsc.VectorSubcoreMesh(core_axis_name="c", subcore_axis_name="s")

    @functools.partial(
        pl.kernel,
        out_type=(
            tuple(jax.ShapeDtypeStruct((n, q), jnp.float32) for _ in range(4))
            + (jax.ShapeDtypeStruct((n, 8), jnp.float32),)
        ),
        mesh=mesh,
        scratch_types=[
            pltpu.VMEM((8, _IW), jnp.int32),
            pltpu.VMEM((8, _IW), jnp.int32),
            pltpu.VMEM((burst, _IW, q), jnp.float32),
            pltpu.VMEM((_CH, q), jnp.float32),
            pltpu.VMEM((_CH, 8), jnp.float32),
            pltpu.VMEM((_IW, 8), jnp.float32),
            pltpu.VMEM_SHARED((npad, q), jnp.float32),
            pltpu.VMEM_SHARED((npad, 8), jnp.float32),
            pltpu.SemaphoreType.DMA,
        ],
        compiler_params=pltpu.CompilerParams(use_tc_tiling_on_sc=False),
    )
    def k(h0_hbm, h1_hbm, h2_hbm, h3_hbm, src_hbm, dst_hbm,
          zrow_hbm, zc8_hbm, ones8_hbm,
          a0_hbm, a1_hbm, a2_hbm, a3_hbm, cnt8_hbm,
          src_v, dst_v, rows_v, buf_v, cbuf_v, ones_v, agg_s, cnt_s, sem):
        c = lax.axis_index("c")
        s = lax.axis_index("s")

        pltpu.sync_copy(zc8_hbm, cbuf_v)
        pltpu.sync_copy(ones8_hbm, ones_v)

        def zero_cnt_chunk(r0):
            pltpu.sync_copy(cbuf_v, cnt_s.at[pl.ds(r0, _CH)])

        @pl.when(c == 0)
        def _():
            _chunk_loop(s, n, zero_cnt_chunk)

        def one_pass(table, out_ref, with_cnt):
            pltpu.sync_copy(zrow_hbm, buf_v)

            def zero_chunk(r0):
                pltpu.sync_copy(buf_v, agg_s.at[pl.ds(r0, _CH)])
            _chunk_loop(s, n, zero_chunk)

            plsc.subcore_barrier()

            def it(i, carry):
                r0 = s * rpt + i * 8
                pltpu.sync_copy(src_hbm.at[pl.ds(r0, 8)], src_v)
                pltpu.sync_copy(dst_hbm.at[pl.ds(r0, 8)], dst_v)
                cps = [pltpu.async_copy(
                    table.at[src_v.at[j]], rows_v.at[j], sem)
                    for j in range(burst)]
                for cp in cps:
                    cp.wait()
                for j in range(burst):
                    pltpu.sync_copy(rows_v.at[j],
                                    agg_s.at[dst_v.at[j]], add=True)
                    if with_cnt:
                        pltpu.sync_copy(ones_v,
                                        cnt_s.at[dst_v.at[j]], add=True)
                return carry
            lax.fori_loop(0, rpt // 8, it, 0)

            plsc.subcore_barrier()

            def wb_chunk(r0):
                pltpu.sync_copy(agg_s.at[pl.ds(r0, _CH)], buf_v)
                pltpu.sync_copy(buf_v, out_ref.at[pl.ds(r0, _CH)])
            _chunk_loop(s, n, wb_chunk)

            plsc.subcore_barrier()

        @pl.when(c == 0)
        def _():
            one_pass(h0_hbm, a0_hbm, True)
            one_pass(h1_hbm, a1_hbm, False)

            def wb_cnt_chunk(r0):
                pltpu.sync_copy(cnt_s.at[pl.ds(r0, _CH)], cbuf_v)
                pltpu.sync_copy(cbuf_v, cnt8_hbm.at[pl.ds(r0, _CH)])
            _chunk_loop(s, n, wb_cnt_chunk)

        @pl.when(c == 1)
        def _():
            one_pass(h2_hbm, a2_hbm, False)
            one_pass(h3_hbm, a3_hbm, False)

    return k(*hs, src2d, dst2d, zrow, zc8, ones8)


def _sc_agg_narrow(p, src2d, dst2d, zrow):
    n, w = p.shape
    rows_total = src2d.shape[0]
    rpc = rows_total // _NC
    rpt = rpc // _NS
    burst = 8
    it_n = rpt // burst
    npad = n + 8
    mesh = plsc.VectorSubcoreMesh(core_axis_name="c", subcore_axis_name="s")

    @functools.partial(
        pl.kernel,
        out_type=(
            jax.ShapeDtypeStruct((n, w), jnp.float32),
            jax.ShapeDtypeStruct((n, w), jnp.float32),
        ),
        mesh=mesh,
        scratch_types=[
            pltpu.VMEM((burst, _IW), jnp.int32),
            pltpu.VMEM((burst, _IW), jnp.int32),
            pltpu.VMEM((burst, _IW, w), jnp.float32),
            pltpu.VMEM((_CH, w), jnp.float32),
            pltpu.VMEM_SHARED((npad, w), jnp.float32),
            pltpu.SemaphoreType.DMA,
        ],
        compiler_params=pltpu.CompilerParams(use_tc_tiling_on_sc=False),
    )
    def k(p_hbm, src_hbm, dst_hbm, zrow_hbm, a2a_hbm, a2b_hbm,
          src_v, dst_v, rows_v, buf_v, agg_s, sem):
        c = lax.axis_index("c")
        s = lax.axis_index("s")

        pltpu.sync_copy(zrow_hbm, buf_v)

        def zero_chunk(r0):
            pltpu.sync_copy(buf_v, agg_s.at[pl.ds(r0, _CH)])
        _chunk_loop(s, n, zero_chunk)

        plsc.subcore_barrier()

        def it(i, carry):
            r0 = c * rpc + s * rpt + i * burst
            pltpu.sync_copy(src_hbm.at[pl.ds(r0, burst)], src_v)
            pltpu.sync_copy(dst_hbm.at[pl.ds(r0, burst)], dst_v)
            cps = [pltpu.async_copy(p_hbm.at[src_v.at[j]], rows_v.at[j], sem)
                   for j in range(burst)]
            for cp in cps:
                cp.wait()
            for j in range(burst):
                pltpu.sync_copy(rows_v.at[j], agg_s.at[dst_v.at[j]], add=True)
            return carry
        lax.fori_loop(0, it_n, it, 0)

        plsc.subcore_barrier()

        def wb_chunk_a(r0):
            pltpu.sync_copy(agg_s.at[pl.ds(r0, _CH)], buf_v)
            pltpu.sync_copy(buf_v, a2a_hbm.at[pl.ds(r0, _CH)])

        def wb_chunk_b(r0):
            pltpu.sync_copy(agg_s.at[pl.ds(r0, _CH)], buf_v)
            pltpu.sync_copy(buf_v, a2b_hbm.at[pl.ds(r0, _CH)])

        @pl.when(c == 0)
        def _():
            _chunk_loop(s, n, wb_chunk_a)

        @pl.when(c == 1)
        def _():
            _chunk_loop(s, n, wb_chunk_b)

    return k(p, src2d, dst2d, zrow)


def kernel(x, edge_index, proj_W, proj_b, l1_Wl, l1_bl, l1_Wr,
           l2_Wl, l2_bl, l2_Wr):
    n, d = x.shape
    e = edge_index.shape[1]
    h = l1_Wl.shape[0]
    c = l2_Wl.shape[0]
    cp = 64
    half = d // 2

    rows_needed = -(-e // _IW)
    rows_total = ((rows_needed + 255) // 256) * 256
    epad = rows_total * _IW
    src = edge_index[0]
    dst = edge_index[1]
    srcp = jnp.concatenate(
        [src, jnp.zeros((epad - e,), jnp.int32)]).reshape(rows_total, _IW)
    dstp = jnp.concatenate(
        [dst, jnp.full((epad - e,), n, jnp.int32)]).reshape(rows_total, _IW)

    q = d // 4
    wpT = proj_W.T
    bp = proj_b.reshape(1, d)
    wlT = l1_Wl.T
    wls = [wlT[i * q:(i + 1) * q] for i in range(4)]
    bl = l1_bl.reshape(1, h)
    wrT = l1_Wr.T
    w2 = jnp.zeros((h, cp), jnp.float32).at[:, :c].set(l2_Wl.T)
    wr2 = jnp.zeros((h, cp), jnp.float32).at[:, :c].set(l2_Wr.T)
    b2 = jnp.zeros((1, cp), jnp.float32).at[:, :c].set(l2_bl.reshape(1, c))

    zrow = jnp.zeros((_CH, q), jnp.float32)
    zc8 = jnp.zeros((_CH, 8), jnp.float32)
    ones8 = jnp.ones((_IW, 8), jnp.float32)
    z64 = jnp.zeros((_CH, cp), jnp.float32)

    hs = _tc1(x, wpT, bp)
    a0, a1, a2, a3, cnt8 = _sc_agg_wide(hs, srcp, dstp, zrow, zc8, ones8)
    h1full, p = _tc2([a0, a1, a2, a3], cnt8, x, wls, bl, wrT, w2)
    a2a, a2b = _sc_agg_narrow(p, srcp, dstp, z64)
    out = _tc3(a2a, a2b, cnt8, h1full, wr2, b2, c)
    return out[:, :c]


# speedup vs baseline: 4.3977x; 1.0446x over previous
import functools

import jax
import jax.numpy as jnp
from jax import lax
from jax.experimental import pallas as pl
from jax.experimental.pallas import tpu as pltpu
from jax.experimental.pallas import tpu_sc as plsc

_NC = 2
_NS = 16
_IW = 128


def _tc1_body(x_ref, w_ref, b_ref, h0_ref, h1_ref, h2_ref, h3_ref):
    h = jnp.dot(x_ref[...], w_ref[...], preferred_element_type=jnp.float32)
    h = jnp.maximum(h + b_ref[...], 0.0)
    q = h.shape[1] // 4
    h0_ref[...] = h[:, 0 * q:1 * q]
    h1_ref[...] = h[:, 1 * q:2 * q]
    h2_ref[...] = h[:, 2 * q:3 * q]
    h3_ref[...] = h[:, 3 * q:4 * q]


def _tc1(x, wT, b):
    n, d = x.shape
    q = d // 4
    br = 2000
    return pl.pallas_call(
        _tc1_body,
        grid=(n // br,),
        in_specs=[
            pl.BlockSpec((br, d), lambda i: (i, 0)),
            pl.BlockSpec((d, d), lambda i: (0, 0)),
            pl.BlockSpec((1, d), lambda i: (0, 0)),
        ],
        out_specs=[pl.BlockSpec((br, q), lambda i: (i, 0))] * 4,
        out_shape=[jax.ShapeDtypeStruct((n, q), jnp.float32)] * 4,
    )(x, wT, b)


def _tc2_body(a0_ref, a1_ref, a2_ref, a3_ref, cnt_ref, x_ref,
              wl0_ref, wl1_ref, wl2_ref, wl3_ref, bl_ref,
              wr_ref, w2_ref, h1_ref, p_ref):
    inv = 1.0 / jnp.maximum(cnt_ref[...][:, :1], 1.0)
    aggmm = (jnp.dot(a0_ref[...], wl0_ref[...], preferred_element_type=jnp.float32)
             + jnp.dot(a1_ref[...], wl1_ref[...], preferred_element_type=jnp.float32)
             + jnp.dot(a2_ref[...], wl2_ref[...], preferred_element_type=jnp.float32)
             + jnp.dot(a3_ref[...], wl3_ref[...], preferred_element_type=jnp.float32))
    h1 = (inv * aggmm + bl_ref[...]
          + jnp.dot(x_ref[...], wr_ref[...], preferred_element_type=jnp.float32))
    h1_ref[...] = h1
    p_ref[...] = jnp.dot(h1, w2_ref[...], preferred_element_type=jnp.float32)


def _tc2(aggs, cnt8, x, wls, bl, wrT, w2):
    n, q = aggs[0].shape
    d = x.shape[1]
    h = wrT.shape[1]
    cp = w2.shape[1]
    br = 2000
    return pl.pallas_call(
        _tc2_body,
        grid=(n // br,),
        in_specs=(
            [pl.BlockSpec((br, q), lambda i: (i, 0))] * 4
            + [
                pl.BlockSpec((br, 8), lambda i: (i, 0)),
                pl.BlockSpec((br, d), lambda i: (i, 0)),
            ]
            + [pl.BlockSpec((q, h), lambda i: (0, 0))] * 4
            + [
                pl.BlockSpec((1, h), lambda i: (0, 0)),
                pl.BlockSpec((d, h), lambda i: (0, 0)),
                pl.BlockSpec((h, cp), lambda i: (0, 0)),
            ]
        ),
        out_specs=[
            pl.BlockSpec((br, h), lambda i: (i, 0)),
            pl.BlockSpec((br, cp), lambda i: (i, 0)),
        ],
        out_shape=[
            jax.ShapeDtypeStruct((n, h), jnp.float32),
            jax.ShapeDtypeStruct((n, cp), jnp.float32),
        ],
    )(*aggs, cnt8, x, *wls, bl, wrT, w2)


def _tc3_body(c_real, a2a_ref, a2b_ref, cnt_ref, h1_ref, wr2_ref, b2_ref, o_ref):
    inv = 1.0 / jnp.maximum(cnt_ref[...][:, :1], 1.0)
    logits = (inv * (a2a_ref[...] + a2b_ref[...]) + b2_ref[...]
              + jnp.dot(h1_ref[...], wr2_ref[...], preferred_element_type=jnp.float32))
    col = lax.broadcasted_iota(jnp.int32, logits.shape, 1)
    logits = jnp.where(col < c_real, logits, -1e30)
    m = jnp.max(logits, axis=1, keepdims=True)
    ls = jnp.log(jnp.sum(jnp.exp(logits - m), axis=1, keepdims=True))
    o_ref[...] = logits - m - ls


def _tc3(a2a, a2b, cnt8, h1, wr2, b2, c_real):
    n, cp = a2a.shape
    h = h1.shape[1]
    br = 2000
    return pl.pallas_call(
        functools.partial(_tc3_body, c_real),
        grid=(n // br,),
        in_specs=[
            pl.BlockSpec((br, cp), lambda i: (i, 0)),
            pl.BlockSpec((br, cp), lambda i: (i, 0)),
            pl.BlockSpec((br, 8), lambda i: (i, 0)),
            pl.BlockSpec((br, h), lambda i: (i, 0)),
            pl.BlockSpec((h, cp), lambda i: (0, 0)),
            pl.BlockSpec((1, cp), lambda i: (0, 0)),
        ],
        out_specs=pl.BlockSpec((br, cp), lambda i: (i, 0)),
        out_shape=jax.ShapeDtypeStruct((n, cp), jnp.float32),
    )(a2a, a2b, cnt8, h1, wr2, b2)


_CH = 40


def _chunk_loop(s, n, fn):
    nch = n // _CH
    bound = nch // _NS + jnp.where(s < (nch % _NS), 1, 0).astype(jnp.int32)

    def it(k, carry):
        fn((s + k * _NS) * _CH)
        return carry
    lax.fori_loop(0, bound, it, 0)


def _fire_idx_gather(src_hbm, dst_hbm, table, src_v, dst_v, rows_v,
                     sem_g, bi, bsz, r0):
    pltpu.sync_copy(src_hbm.at[pl.ds(r0, bsz)], src_v.at[pl.ds(bi * bsz, bsz)])
    pltpu.sync_copy(dst_hbm.at[pl.ds(r0, bsz)], dst_v.at[pl.ds(bi * bsz, bsz)])
    hw = _IW // 2
    for j in range(bsz):
        for g in range(2):
            pltpu.async_copy(
                table.at[src_v.at[bi * bsz + j].at[pl.ds(g * hw, hw)]],
                rows_v.at[bi * bsz + j].at[pl.ds(g * hw, hw)], sem_g)


def _edge_pipeline(src_hbm, dst_hbm, table, agg_s, src_v, dst_v, rows_v,
                   sem_g, sem_s, base_row, nrows, bsz,
                   cnt_s=None, ones_v=None):
    nblk = nrows // bsz
    pairs = nblk // 2

    hw = _IW // 2

    def wait_g(bi):
        for j in range(bsz):
            for g in range(2):
                pltpu.make_async_copy(
                    table.at[src_v.at[bi * bsz + j].at[pl.ds(g * hw, hw)]],
                    rows_v.at[bi * bsz + j].at[pl.ds(g * hw, hw)], sem_g).wait()

    def fire_s(bi):
        for j in range(bsz):
            pltpu.async_copy(rows_v.at[bi * bsz + j],
                             agg_s.at[dst_v.at[bi * bsz + j]], sem_s, add=True)
            if cnt_s is not None:
                pltpu.async_copy(ones_v, cnt_s.at[dst_v.at[bi * bsz + j]],
                                 sem_s, add=True)

    def wait_s(bi):
        for j in range(bsz):
            pltpu.make_async_copy(rows_v.at[bi * bsz + j],
                                  agg_s.at[dst_v.at[bi * bsz + j]], sem_s).wait()
            if cnt_s is not None:
                pltpu.make_async_copy(ones_v, cnt_s.at[dst_v.at[bi * bsz + j]],
                                      sem_s).wait()

    def fire_g(bi, blk):
        _fire_idx_gather(src_hbm, dst_hbm, table, src_v, dst_v, rows_v,
                         sem_g, bi, bsz, base_row + blk * bsz)

    fire_g(0, 0)

    def body(t, carry):
        blk0 = 2 * t

        @pl.when(t > 0)
        def _():
            wait_s(1)
        fire_g(1, blk0 + 1)
        wait_g(0)
        fire_s(0)
        wait_s(0)
        @pl.when(t < pairs - 1)
        def _():
            fire_g(0, blk0 + 2)
        wait_g(1)
        fire_s(1)
        return carry
    lax.fori_loop(0, pairs, body, 0)
    wait_s(1)


def _sc_agg_wide(hs, src2d, dst2d, zrow, zc8, ones8):
    n, q = hs[0].shape
    rows_total = src2d.shape[0]
    rpt = rows_total // _NS
    bsz = 4
    npad = n + _IW
    mesh = plsc.VectorSubcoreMesh(core_axis_name="c", subcore_axis_name="s")

    @functools.partial(
        pl.kernel,
        out_type=(
            tuple(jax.ShapeDtypeStruct((n, q), jnp.float32) for _ in range(4))
            + (jax.ShapeDtypeStruct((n, 8), jnp.float32),)
        ),
        mesh=mesh,
        scratch_types=[
            pltpu.VMEM((2 * bsz, _IW), jnp.int32),
            pltpu.VMEM((2 * bsz, _IW), jnp.int32),
            pltpu.VMEM((2 * bsz, _IW, q), jnp.float32),
            pltpu.VMEM((_CH, q), jnp.float32),
            pltpu.VMEM((_CH, 8), jnp.float32),
            pltpu.VMEM((_IW, 8), jnp.float32),
            pltpu.VMEM_SHARED((npad, q), jnp.float32),
            pltpu.VMEM_SHARED((npad, 8), jnp.float32),
            pltpu.SemaphoreType.DMA,
            pltpu.SemaphoreType.DMA,
        ],
        compiler_params=pltpu.CompilerParams(use_tc_tiling_on_sc=False),
    )
    def k(h0_hbm, h1_hbm, h2_hbm, h3_hbm, src_hbm, dst_hbm,
          zrow_hbm, zc8_hbm, ones8_hbm,
          a0_hbm, a1_hbm, a2_hbm, a3_hbm, cnt8_hbm,
          src_v, dst_v, rows_v, buf_v, cbuf_v, ones_v, agg_s, cnt_s,
          sem_g, sem_s):
        c = lax.axis_index("c")
        s = lax.axis_index("s")

        pltpu.sync_copy(zc8_hbm, cbuf_v)
        pltpu.sync_copy(ones8_hbm, ones_v)

        def zero_cnt_chunk(r0):
            pltpu.sync_copy(cbuf_v, cnt_s.at[pl.ds(r0, _CH)])

        @pl.when(c == 0)
        def _():
            _chunk_loop(s, n, zero_cnt_chunk)

        def one_pass(table, out_ref, with_cnt):
            pltpu.sync_copy(zrow_hbm, buf_v)

            def zero_chunk(r0):
                pltpu.sync_copy(buf_v, agg_s.at[pl.ds(r0, _CH)])
            _chunk_loop(s, n, zero_chunk)

            plsc.subcore_barrier()

            _edge_pipeline(src_hbm, dst_hbm, table, agg_s,
                           src_v, dst_v, rows_v, sem_g, sem_s,
                           s * rpt, rpt, bsz,
                           cnt_s=cnt_s if with_cnt else None, ones_v=ones_v)

            plsc.subcore_barrier()

            def wb_chunk(r0):
                pltpu.sync_copy(agg_s.at[pl.ds(r0, _CH)], buf_v)
                pltpu.sync_copy(buf_v, out_ref.at[pl.ds(r0, _CH)])
            _chunk_loop(s, n, wb_chunk)

            plsc.subcore_barrier()

        @pl.when(c == 0)
        def _():
            one_pass(h0_hbm, a0_hbm, True)
            one_pass(h1_hbm, a1_hbm, False)

            def wb_cnt_chunk(r0):
                pltpu.sync_copy(cnt_s.at[pl.ds(r0, _CH)], cbuf_v)
                pltpu.sync_copy(cbuf_v, cnt8_hbm.at[pl.ds(r0, _CH)])
            _chunk_loop(s, n, wb_cnt_chunk)

        @pl.when(c == 1)
        def _():
            one_pass(h2_hbm, a2_hbm, False)
            one_pass(h3_hbm, a3_hbm, False)

    return k(*hs, src2d, dst2d, zrow, zc8, ones8)


def _sc_agg_narrow(p, src2d, dst2d, zrow):
    n, w = p.shape
    rows_total = src2d.shape[0]
    rpc = rows_total // _NC
    rpt = rpc // _NS
    bsz = 4
    npad = n + _IW
    mesh = plsc.VectorSubcoreMesh(core_axis_name="c", subcore_axis_name="s")

    @functools.partial(
        pl.kernel,
        out_type=(
            jax.ShapeDtypeStruct((n, w), jnp.float32),
            jax.ShapeDtypeStruct((n, w), jnp.float32),
        ),
        mesh=mesh,
        scratch_types=[
            pltpu.VMEM((2 * bsz, _IW), jnp.int32),
            pltpu.VMEM((2 * bsz, _IW), jnp.int32),
            pltpu.VMEM((2 * bsz, _IW, w), jnp.float32),
            pltpu.VMEM((_CH, w), jnp.float32),
            pltpu.VMEM_SHARED((npad, w), jnp.float32),
            pltpu.SemaphoreType.DMA,
            pltpu.SemaphoreType.DMA,
        ],
        compiler_params=pltpu.CompilerParams(use_tc_tiling_on_sc=False),
    )
    def k(p_hbm, src_hbm, dst_hbm, zrow_hbm, a2a_hbm, a2b_hbm,
          src_v, dst_v, rows_v, buf_v, agg_s, sem_g, sem_s):
        c = lax.axis_index("c")
        s = lax.axis_index("s")

        pltpu.sync_copy(zrow_hbm, buf_v)

        def zero_chunk(r0):
            pltpu.sync_copy(buf_v, agg_s.at[pl.ds(r0, _CH)])
        _chunk_loop(s, n, zero_chunk)

        plsc.subcore_barrier()

        _edge_pipeline(src_hbm, dst_hbm, p_hbm, agg_s,
                       src_v, dst_v, rows_v, sem_g, sem_s,
                       c * rpc + s * rpt, rpt, bsz)

        plsc.subcore_barrier()

        def wb_chunk_a(r0):
            pltpu.sync_copy(agg_s.at[pl.ds(r0, _CH)], buf_v)
            pltpu.sync_copy(buf_v, a2a_hbm.at[pl.ds(r0, _CH)])

        def wb_chunk_b(r0):
            pltpu.sync_copy(agg_s.at[pl.ds(r0, _CH)], buf_v)
            pltpu.sync_copy(buf_v, a2b_hbm.at[pl.ds(r0, _CH)])

        @pl.when(c == 0)
        def _():
            _chunk_loop(s, n, wb_chunk_a)

        @pl.when(c == 1)
        def _():
            _chunk_loop(s, n, wb_chunk_b)

    return k(p, src2d, dst2d, zrow)


def kernel(x, edge_index, proj_W, proj_b, l1_Wl, l1_bl, l1_Wr,
           l2_Wl, l2_bl, l2_Wr):
    n, d = x.shape
    e = edge_index.shape[1]
    h = l1_Wl.shape[0]
    c = l2_Wl.shape[0]
    cp = 64
    half = d // 2

    rows_needed = -(-e // _IW)
    rows_total = ((rows_needed + 255) // 256) * 256
    epad = rows_total * _IW
    src = edge_index[0]
    dst = edge_index[1]
    srcp = jnp.concatenate(
        [src, jnp.zeros((epad - e,), jnp.int32)]).reshape(rows_total, _IW)
    pad_dst = n + (jnp.arange(epad - e, dtype=jnp.int32) % _IW)
    dstp = jnp.concatenate([dst, pad_dst]).reshape(rows_total, _IW)

    q = d // 4
    wpT = proj_W.T
    bp = proj_b.reshape(1, d)
    wlT = l1_Wl.T
    wls = [wlT[i * q:(i + 1) * q] for i in range(4)]
    bl = l1_bl.reshape(1, h)
    wrT = l1_Wr.T
    w2 = jnp.zeros((h, cp), jnp.float32).at[:, :c].set(l2_Wl.T)
    wr2 = jnp.zeros((h, cp), jnp.float32).at[:, :c].set(l2_Wr.T)
    b2 = jnp.zeros((1, cp), jnp.float32).at[:, :c].set(l2_bl.reshape(1, c))

    zrow = jnp.zeros((_CH, q), jnp.float32)
    zc8 = jnp.zeros((_CH, 8), jnp.float32)
    ones8 = jnp.ones((_IW, 8), jnp.float32)
    z64 = jnp.zeros((_CH, cp), jnp.float32)

    hs = _tc1(x, wpT, bp)
    a0, a1, a2, a3, cnt8 = _sc_agg_wide(hs, srcp, dstp, zrow, zc8, ones8)
    h1full, p = _tc2([a0, a1, a2, a3], cnt8, x, wls, bl, wrT, w2)
    a2a, a2b = _sc_agg_narrow(p, srcp, dstp, z64)
    out = _tc3(a2a, a2b, cnt8, h1full, wr2, b2, c)
    return out[:, :c]


# speedup vs baseline: 5.8939x; 1.3402x over previous
import functools

import jax
import jax.numpy as jnp
from jax import lax
from jax.experimental import pallas as pl
from jax.experimental.pallas import tpu as pltpu
from jax.experimental.pallas import tpu_sc as plsc

_NC = 2
_NS = 16
_IW = 128


def _tc1_body(x_ref, w_ref, b_ref, h0_ref, h1_ref, h2_ref, h3_ref):
    h = jnp.dot(x_ref[...], w_ref[...], preferred_element_type=jnp.float32)
    h = jnp.maximum(h + b_ref[...], 0.0)
    q = h.shape[1] // 4
    h0_ref[...] = h[:, 0 * q:1 * q].astype(jnp.bfloat16)
    h1_ref[...] = h[:, 1 * q:2 * q].astype(jnp.bfloat16)
    h2_ref[...] = h[:, 2 * q:3 * q].astype(jnp.bfloat16)
    h3_ref[...] = h[:, 3 * q:4 * q].astype(jnp.bfloat16)


def _tc1(x, wT, b):
    n, d = x.shape
    q = d // 4
    br = 2000
    return pl.pallas_call(
        _tc1_body,
        grid=(n // br,),
        in_specs=[
            pl.BlockSpec((br, d), lambda i: (i, 0)),
            pl.BlockSpec((d, d), lambda i: (0, 0)),
            pl.BlockSpec((1, d), lambda i: (0, 0)),
        ],
        out_specs=[pl.BlockSpec((br, q), lambda i: (i, 0))] * 4,
        out_shape=[jax.ShapeDtypeStruct((n, q), jnp.bfloat16)] * 4,
    )(x, wT, b)


def _tc2_body(a0_ref, a1_ref, a2_ref, a3_ref, cnt_ref, x_ref,
              wl0_ref, wl1_ref, wl2_ref, wl3_ref, bl_ref,
              wr_ref, w2_ref, h1_ref, p_ref):
    inv = 1.0 / jnp.maximum(cnt_ref[...][:, :1], 1.0)
    aggmm = (jnp.dot(a0_ref[...].astype(jnp.float32), wl0_ref[...],
                     preferred_element_type=jnp.float32)
             + jnp.dot(a1_ref[...].astype(jnp.float32), wl1_ref[...],
                       preferred_element_type=jnp.float32)
             + jnp.dot(a2_ref[...].astype(jnp.float32), wl2_ref[...],
                       preferred_element_type=jnp.float32)
             + jnp.dot(a3_ref[...].astype(jnp.float32), wl3_ref[...],
                       preferred_element_type=jnp.float32))
    h1 = (inv * aggmm + bl_ref[...]
          + jnp.dot(x_ref[...], wr_ref[...], preferred_element_type=jnp.float32))
    h1_ref[...] = h1
    p_ref[...] = jnp.dot(h1, w2_ref[...], preferred_element_type=jnp.float32)


def _tc2(aggs, cnt8, x, wls, bl, wrT, w2):
    n, q = aggs[0].shape
    d = x.shape[1]
    h = wrT.shape[1]
    cp = w2.shape[1]
    br = 2000
    return pl.pallas_call(
        _tc2_body,
        grid=(n // br,),
        in_specs=(
            [pl.BlockSpec((br, q), lambda i: (i, 0))] * 4
            + [
                pl.BlockSpec((br, 8), lambda i: (i, 0)),
                pl.BlockSpec((br, d), lambda i: (i, 0)),
            ]
            + [pl.BlockSpec((q, h), lambda i: (0, 0))] * 4
            + [
                pl.BlockSpec((1, h), lambda i: (0, 0)),
                pl.BlockSpec((d, h), lambda i: (0, 0)),
                pl.BlockSpec((h, cp), lambda i: (0, 0)),
            ]
        ),
        out_specs=[
            pl.BlockSpec((br, h), lambda i: (i, 0)),
            pl.BlockSpec((br, cp), lambda i: (i, 0)),
        ],
        out_shape=[
            jax.ShapeDtypeStruct((n, h), jnp.float32),
            jax.ShapeDtypeStruct((n, cp), jnp.float32),
        ],
    )(*aggs, cnt8, x, *wls, bl, wrT, w2)


def _tc3_body(a2a_ref, a2b_ref, cnt_ref, h1_ref, wr2_ref, b2_ref, o_ref):
    inv = 1.0 / jnp.maximum(cnt_ref[...][:, :1], 1.0)
    logits = (inv * (a2a_ref[...] + a2b_ref[...]) + b2_ref[...]
              + jnp.dot(h1_ref[...], wr2_ref[...], preferred_element_type=jnp.float32))
    m = jnp.max(logits, axis=1, keepdims=True)
    ls = jnp.log(jnp.sum(jnp.exp(logits - m), axis=1, keepdims=True))
    o_ref[...] = logits - m - ls


def _tc3(a2a, a2b, cnt8, h1, wr2, b2):
    n, cp = a2a.shape
    h = h1.shape[1]
    br = 2000
    return pl.pallas_call(
        _tc3_body,
        grid=(n // br,),
        in_specs=[
            pl.BlockSpec((br, cp), lambda i: (i, 0)),
            pl.BlockSpec((br, cp), lambda i: (i, 0)),
            pl.BlockSpec((br, 8), lambda i: (i, 0)),
            pl.BlockSpec((br, h), lambda i: (i, 0)),
            pl.BlockSpec((h, cp), lambda i: (0, 0)),
            pl.BlockSpec((1, cp), lambda i: (0, 0)),
        ],
        out_specs=pl.BlockSpec((br, cp), lambda i: (i, 0)),
        out_shape=jax.ShapeDtypeStruct((n, cp), jnp.float32),
    )(a2a, a2b, cnt8, h1, wr2, b2)


_CH = 40


def _chunk_loop(s, n, fn):
    nch = n // _CH
    bound = nch // _NS + jnp.where(s < (nch % _NS), 1, 0).astype(jnp.int32)

    def it(k, carry):
        fn((s + k * _NS) * _CH)
        return carry
    lax.fori_loop(0, bound, it, 0)


def _fire_idx_gather(src_hbm, dst_hbm, table, src_v, dst_v, rows_v,
                     sem_g, bi, bsz, r0):
    pltpu.sync_copy(src_hbm.at[pl.ds(r0, bsz)], src_v.at[pl.ds(bi * bsz, bsz)])
    pltpu.sync_copy(dst_hbm.at[pl.ds(r0, bsz)], dst_v.at[pl.ds(bi * bsz, bsz)])
    for j in range(bsz):
        pltpu.async_copy(table.at[src_v.at[bi * bsz + j]],
                         rows_v.at[bi * bsz + j], sem_g)


def _edge_pipeline(src_hbm, dst_hbm, table, agg_s, src_v, dst_v, rows_v,
                   sem_g, sem_s, base_row, nrows, bsz,
                   cnt_s=None, ones_v=None):
    nblk = nrows // bsz
    pairs = nblk // 2

    def wait_g(bi):
        for j in range(bsz):
            pltpu.make_async_copy(table.at[src_v.at[bi * bsz + j]],
                                  rows_v.at[bi * bsz + j], sem_g).wait()

    def fire_s(bi):
        for j in range(bsz):
            pltpu.async_copy(rows_v.at[bi * bsz + j],
                             agg_s.at[dst_v.at[bi * bsz + j]], sem_s, add=True)
            if cnt_s is not None:
                pltpu.async_copy(ones_v, cnt_s.at[dst_v.at[bi * bsz + j]],
                                 sem_s, add=True)

    def wait_s(bi):
        for j in range(bsz):
            pltpu.make_async_copy(rows_v.at[bi * bsz + j],
                                  agg_s.at[dst_v.at[bi * bsz + j]], sem_s).wait()
            if cnt_s is not None:
                pltpu.make_async_copy(ones_v, cnt_s.at[dst_v.at[bi * bsz + j]],
                                      sem_s).wait()

    def fire_g(bi, blk):
        _fire_idx_gather(src_hbm, dst_hbm, table, src_v, dst_v, rows_v,
                         sem_g, bi, bsz, base_row + blk * bsz)

    fire_g(0, 0)

    def body(t, carry):
        blk0 = 2 * t

        @pl.when(t > 0)
        def _():
            wait_s(1)
        fire_g(1, blk0 + 1)
        wait_g(0)
        fire_s(0)
        wait_s(0)
        @pl.when(t < pairs - 1)
        def _():
            fire_g(0, blk0 + 2)
        wait_g(1)
        fire_s(1)
        return carry
    lax.fori_loop(0, pairs, body, 0)
    wait_s(1)


def _sc_agg_wide(hs, src2d, dst2d, zrow, zc8, ones8):
    n, q = hs[0].shape
    rows_total = src2d.shape[0]
    rpt = rows_total // _NS
    bsz = 4
    npad = n + _IW
    mesh = plsc.VectorSubcoreMesh(core_axis_name="c", subcore_axis_name="s")

    @functools.partial(
        pl.kernel,
        out_type=(
            tuple(jax.ShapeDtypeStruct((n, q), jnp.bfloat16) for _ in range(4))
            + (jax.ShapeDtypeStruct((n, 8), jnp.float32),)
        ),
        mesh=mesh,
        scratch_types=[
            pltpu.VMEM((2 * bsz, _IW), jnp.int32),
            pltpu.VMEM((2 * bsz, _IW), jnp.int32),
            pltpu.VMEM((2 * bsz, _IW, q), jnp.bfloat16),
            pltpu.VMEM((_CH, q), jnp.bfloat16),
            pltpu.VMEM((_CH, 8), jnp.float32),
            pltpu.VMEM((_IW, 8), jnp.float32),
            pltpu.VMEM_SHARED((npad, q), jnp.bfloat16),
            pltpu.VMEM_SHARED((npad, 8), jnp.float32),
            pltpu.SemaphoreType.DMA,
            pltpu.SemaphoreType.DMA,
        ],
        compiler_params=pltpu.CompilerParams(use_tc_tiling_on_sc=False),
    )
    def k(h0_hbm, h1_hbm, h2_hbm, h3_hbm, src_hbm, dst_hbm,
          zrow_hbm, zc8_hbm, ones8_hbm,
          a0_hbm, a1_hbm, a2_hbm, a3_hbm, cnt8_hbm,
          src_v, dst_v, rows_v, buf_v, cbuf_v, ones_v, agg_s, cnt_s,
          sem_g, sem_s):
        c = lax.axis_index("c")
        s = lax.axis_index("s")

        pltpu.sync_copy(zc8_hbm, cbuf_v)
        pltpu.sync_copy(ones8_hbm, ones_v)

        def zero_cnt_chunk(r0):
            pltpu.sync_copy(cbuf_v, cnt_s.at[pl.ds(r0, _CH)])

        @pl.when(c == 0)
        def _():
            _chunk_loop(s, n, zero_cnt_chunk)

        def one_pass(table, out_ref, with_cnt):
            pltpu.sync_copy(zrow_hbm, buf_v)

            def zero_chunk(r0):
                pltpu.sync_copy(buf_v, agg_s.at[pl.ds(r0, _CH)])
            _chunk_loop(s, n, zero_chunk)

            plsc.subcore_barrier()

            _edge_pipeline(src_hbm, dst_hbm, table, agg_s,
                           src_v, dst_v, rows_v, sem_g, sem_s,
                           s * rpt, rpt, bsz,
                           cnt_s=cnt_s if with_cnt else None, ones_v=ones_v)

            plsc.subcore_barrier()

            def wb_chunk(r0):
                pltpu.sync_copy(agg_s.at[pl.ds(r0, _CH)], buf_v)
                pltpu.sync_copy(buf_v, out_ref.at[pl.ds(r0, _CH)])
            _chunk_loop(s, n, wb_chunk)

            plsc.subcore_barrier()

        @pl.when(c == 0)
        def _():
            one_pass(h0_hbm, a0_hbm, True)
            one_pass(h1_hbm, a1_hbm, False)

            def wb_cnt_chunk(r0):
                pltpu.sync_copy(cnt_s.at[pl.ds(r0, _CH)], cbuf_v)
                pltpu.sync_copy(cbuf_v, cnt8_hbm.at[pl.ds(r0, _CH)])
            _chunk_loop(s, n, wb_cnt_chunk)

        @pl.when(c == 1)
        def _():
            one_pass(h2_hbm, a2_hbm, False)
            one_pass(h3_hbm, a3_hbm, False)

    return k(*hs, src2d, dst2d, zrow, zc8, ones8)


def _sc_agg_narrow(p, src2d, dst2d, zrow):
    n, w = p.shape
    rows_total = src2d.shape[0]
    rpc = rows_total // _NC
    rpt = rpc // _NS
    bsz = 4
    npad = n + _IW
    mesh = plsc.VectorSubcoreMesh(core_axis_name="c", subcore_axis_name="s")

    @functools.partial(
        pl.kernel,
        out_type=(
            jax.ShapeDtypeStruct((n, w), jnp.float32),
            jax.ShapeDtypeStruct((n, w), jnp.float32),
        ),
        mesh=mesh,
        scratch_types=[
            pltpu.VMEM((2 * bsz, _IW), jnp.int32),
            pltpu.VMEM((2 * bsz, _IW), jnp.int32),
            pltpu.VMEM((2 * bsz, _IW, w), jnp.float32),
            pltpu.VMEM((_CH, w), jnp.float32),
            pltpu.VMEM_SHARED((npad, w), jnp.float32),
            pltpu.SemaphoreType.DMA,
            pltpu.SemaphoreType.DMA,
        ],
        compiler_params=pltpu.CompilerParams(use_tc_tiling_on_sc=False),
    )
    def k(p_hbm, src_hbm, dst_hbm, zrow_hbm, a2a_hbm, a2b_hbm,
          src_v, dst_v, rows_v, buf_v, agg_s, sem_g, sem_s):
        c = lax.axis_index("c")
        s = lax.axis_index("s")

        pltpu.sync_copy(zrow_hbm, buf_v)

        def zero_chunk(r0):
            pltpu.sync_copy(buf_v, agg_s.at[pl.ds(r0, _CH)])
        _chunk_loop(s, n, zero_chunk)

        plsc.subcore_barrier()

        _edge_pipeline(src_hbm, dst_hbm, p_hbm, agg_s,
                       src_v, dst_v, rows_v, sem_g, sem_s,
                       c * rpc + s * rpt, rpt, bsz)

        plsc.subcore_barrier()

        def wb_chunk_a(r0):
            pltpu.sync_copy(agg_s.at[pl.ds(r0, _CH)], buf_v)
            pltpu.sync_copy(buf_v, a2a_hbm.at[pl.ds(r0, _CH)])

        def wb_chunk_b(r0):
            pltpu.sync_copy(agg_s.at[pl.ds(r0, _CH)], buf_v)
            pltpu.sync_copy(buf_v, a2b_hbm.at[pl.ds(r0, _CH)])

        @pl.when(c == 0)
        def _():
            _chunk_loop(s, n, wb_chunk_a)

        @pl.when(c == 1)
        def _():
            _chunk_loop(s, n, wb_chunk_b)

    return k(p, src2d, dst2d, zrow)


def kernel(x, edge_index, proj_W, proj_b, l1_Wl, l1_bl, l1_Wr,
           l2_Wl, l2_bl, l2_Wr):
    n, d = x.shape
    e = edge_index.shape[1]
    h = l1_Wl.shape[0]
    c = l2_Wl.shape[0]
    half = d // 2

    rows_needed = -(-e // _IW)
    rows_total = ((rows_needed + 255) // 256) * 256
    epad = rows_total * _IW
    src = edge_index[0]
    dst = edge_index[1]
    srcp = jnp.concatenate(
        [src, jnp.zeros((epad - e,), jnp.int32)]).reshape(rows_total, _IW)
    pad_dst = n + (jnp.arange(epad - e, dtype=jnp.int32) % _IW)
    dstp = jnp.concatenate([dst, pad_dst]).reshape(rows_total, _IW)

    q = d // 4
    wpT = proj_W.T
    bp = proj_b.reshape(1, d)
    wlT = l1_Wl.T
    wls = [wlT[i * q:(i + 1) * q] for i in range(4)]
    bl = l1_bl.reshape(1, h)
    wrT = l1_Wr.T
    w2 = l2_Wl.T
    wr2 = l2_Wr.T
    b2 = l2_bl.reshape(1, c)

    zrow = jnp.zeros((_CH, q), jnp.bfloat16)
    zc8 = jnp.zeros((_CH, 8), jnp.float32)
    ones8 = jnp.ones((_IW, 8), jnp.float32)
    zp = jnp.zeros((_CH, c), jnp.float32)

    hs = _tc1(x, wpT, bp)
    a0, a1, a2, a3, cnt8 = _sc_agg_wide(hs, srcp, dstp, zrow, zc8, ones8)
    h1full, p = _tc2([a0, a1, a2, a3], cnt8, x, wls, bl, wrT, w2)
    a2a, a2b = _sc_agg_narrow(p, srcp, dstp, zp)
    return _tc3(a2a, a2b, cnt8, h1full, wr2, b2)


# speedup vs baseline: 6.3278x; 1.0736x over previous
import functools

import jax
import jax.numpy as jnp
from jax import lax
from jax.experimental import pallas as pl
from jax.experimental.pallas import tpu as pltpu
from jax.experimental.pallas import tpu_sc as plsc

_NC = 2
_NS = 16
_IW = 128


def _tc1_body(x_ref, w_ref, b_ref, h0_ref, h1_ref, h2_ref, h3_ref):
    h = jnp.dot(x_ref[...], w_ref[...], preferred_element_type=jnp.float32)
    h = jnp.maximum(h + b_ref[...], 0.0)
    q = h.shape[1] // 4
    h0_ref[...] = h[:, 0 * q:1 * q].astype(jnp.bfloat16)
    h1_ref[...] = h[:, 1 * q:2 * q].astype(jnp.bfloat16)
    h2_ref[...] = h[:, 2 * q:3 * q].astype(jnp.bfloat16)
    h3_ref[...] = h[:, 3 * q:4 * q].astype(jnp.bfloat16)


def _tc1(x, wT, b):
    n, d = x.shape
    q = d // 4
    br = 2000
    return pl.pallas_call(
        _tc1_body,
        grid=(n // br,),
        in_specs=[
            pl.BlockSpec((br, d), lambda i: (i, 0)),
            pl.BlockSpec((d, d), lambda i: (0, 0)),
            pl.BlockSpec((1, d), lambda i: (0, 0)),
        ],
        out_specs=[pl.BlockSpec((br, q), lambda i: (i, 0))] * 4,
        out_shape=[jax.ShapeDtypeStruct((n, q), jnp.bfloat16)] * 4,
    )(x, wT, b)


def _tc2_body(a0_ref, a1_ref, a2_ref, a3_ref, cnt_ref, x_ref,
              wl0_ref, wl1_ref, wl2_ref, wl3_ref, bl_ref,
              wr_ref, w2_ref, h1_ref, p_ref):
    inv = 1.0 / jnp.maximum(cnt_ref[...][:, :1], 1.0)
    aggmm = (jnp.dot(a0_ref[...].astype(jnp.float32), wl0_ref[...],
                     preferred_element_type=jnp.float32)
             + jnp.dot(a1_ref[...].astype(jnp.float32), wl1_ref[...],
                       preferred_element_type=jnp.float32)
             + jnp.dot(a2_ref[...].astype(jnp.float32), wl2_ref[...],
                       preferred_element_type=jnp.float32)
             + jnp.dot(a3_ref[...].astype(jnp.float32), wl3_ref[...],
                       preferred_element_type=jnp.float32))
    h1 = (inv * aggmm + bl_ref[...]
          + jnp.dot(x_ref[...], wr_ref[...], preferred_element_type=jnp.float32))
    h1_ref[...] = h1
    p_ref[...] = jnp.dot(h1, w2_ref[...],
                         preferred_element_type=jnp.float32).astype(jnp.bfloat16)


def _tc2(aggs, cnt8, x, wls, bl, wrT, w2):
    n, q = aggs[0].shape
    d = x.shape[1]
    h = wrT.shape[1]
    cp = w2.shape[1]
    br = 2000
    return pl.pallas_call(
        _tc2_body,
        grid=(n // br,),
        in_specs=(
            [pl.BlockSpec((br, q), lambda i: (i, 0))] * 4
            + [
                pl.BlockSpec((br, 8), lambda i: (i, 0)),
                pl.BlockSpec((br, d), lambda i: (i, 0)),
            ]
            + [pl.BlockSpec((q, h), lambda i: (0, 0))] * 4
            + [
                pl.BlockSpec((1, h), lambda i: (0, 0)),
                pl.BlockSpec((d, h), lambda i: (0, 0)),
                pl.BlockSpec((h, cp), lambda i: (0, 0)),
            ]
        ),
        out_specs=[
            pl.BlockSpec((br, h), lambda i: (i, 0)),
            pl.BlockSpec((br, cp), lambda i: (i, 0)),
        ],
        out_shape=[
            jax.ShapeDtypeStruct((n, h), jnp.float32),
            jax.ShapeDtypeStruct((n, cp), jnp.bfloat16),
        ],
    )(*aggs, cnt8, x, *wls, bl, wrT, w2)


def _tc3_body(c_real, a2a_ref, a2b_ref, cnt_ref, h1_ref, wr2_ref, b2_ref, o_ref):
    inv = 1.0 / jnp.maximum(cnt_ref[...][:, :1], 1.0)
    logits = (inv * (a2a_ref[...].astype(jnp.float32)
                     + a2b_ref[...].astype(jnp.float32)) + b2_ref[...]
              + jnp.dot(h1_ref[...], wr2_ref[...], preferred_element_type=jnp.float32))
    col = lax.broadcasted_iota(jnp.int32, logits.shape, 1)
    logits = jnp.where(col < c_real, logits, -1e30)
    m = jnp.max(logits, axis=1, keepdims=True)
    ls = jnp.log(jnp.sum(jnp.exp(logits - m), axis=1, keepdims=True))
    o_ref[...] = logits - m - ls


def _tc3(a2a, a2b, cnt8, h1, wr2, b2, c_real):
    n, cp = a2a.shape
    h = h1.shape[1]
    br = 2000
    return pl.pallas_call(
        functools.partial(_tc3_body, c_real),
        grid=(n // br,),
        in_specs=[
            pl.BlockSpec((br, cp), lambda i: (i, 0)),
            pl.BlockSpec((br, cp), lambda i: (i, 0)),
            pl.BlockSpec((br, 8), lambda i: (i, 0)),
            pl.BlockSpec((br, h), lambda i: (i, 0)),
            pl.BlockSpec((h, cp), lambda i: (0, 0)),
            pl.BlockSpec((1, cp), lambda i: (0, 0)),
        ],
        out_specs=pl.BlockSpec((br, cp), lambda i: (i, 0)),
        out_shape=jax.ShapeDtypeStruct((n, cp), jnp.float32),
    )(a2a, a2b, cnt8, h1, wr2, b2)


_CH = 40


def _chunk_loop(s, n, fn):
    nch = n // _CH
    bound = nch // _NS + jnp.where(s < (nch % _NS), 1, 0).astype(jnp.int32)

    def it(k, carry):
        fn((s + k * _NS) * _CH)
        return carry
    lax.fori_loop(0, bound, it, 0)


def _fire_idx_gather(src_hbm, dst_hbm, table, src_v, dst_v, rows_v,
                     sem_g, bi, bsz, r0):
    pltpu.sync_copy(src_hbm.at[pl.ds(r0, bsz)], src_v.at[pl.ds(bi * bsz, bsz)])
    pltpu.sync_copy(dst_hbm.at[pl.ds(r0, bsz)], dst_v.at[pl.ds(bi * bsz, bsz)])
    for j in range(bsz):
        pltpu.async_copy(table.at[src_v.at[bi * bsz + j]],
                         rows_v.at[bi * bsz + j], sem_g)


def _edge_pipeline(src_hbm, dst_hbm, table, agg_s, src_v, dst_v, rows_v,
                   sem_g, sem_s, base_row, nrows, bsz,
                   cnt_s=None, ones_v=None):
    nblk = nrows // bsz
    pairs = nblk // 2

    def wait_g(bi):
        for j in range(bsz):
            pltpu.make_async_copy(table.at[src_v.at[bi * bsz + j]],
                                  rows_v.at[bi * bsz + j], sem_g).wait()

    def fire_s(bi):
        for j in range(bsz):
            pltpu.async_copy(rows_v.at[bi * bsz + j],
                             agg_s.at[dst_v.at[bi * bsz + j]], sem_s, add=True)
            if cnt_s is not None:
                pltpu.async_copy(ones_v, cnt_s.at[dst_v.at[bi * bsz + j]],
                                 sem_s, add=True)

    def wait_s(bi):
        for j in range(bsz):
            pltpu.make_async_copy(rows_v.at[bi * bsz + j],
                                  agg_s.at[dst_v.at[bi * bsz + j]], sem_s).wait()
            if cnt_s is not None:
                pltpu.make_async_copy(ones_v, cnt_s.at[dst_v.at[bi * bsz + j]],
                                      sem_s).wait()

    def fire_g(bi, blk):
        _fire_idx_gather(src_hbm, dst_hbm, table, src_v, dst_v, rows_v,
                         sem_g, bi, bsz, base_row + blk * bsz)

    fire_g(0, 0)

    def body(t, carry):
        blk0 = 2 * t

        @pl.when(t > 0)
        def _():
            wait_s(1)
        fire_g(1, blk0 + 1)
        wait_g(0)
        fire_s(0)
        wait_s(0)
        @pl.when(t < pairs - 1)
        def _():
            fire_g(0, blk0 + 2)
        wait_g(1)
        fire_s(1)
        return carry
    lax.fori_loop(0, pairs, body, 0)
    wait_s(1)


def _sc_agg_wide(hs, src2d, dst2d, zrow, zc8, ones8):
    n, q = hs[0].shape
    rows_total = src2d.shape[0]
    rpt = rows_total // _NS
    bsz = 4
    npad = n + _IW
    mesh = plsc.VectorSubcoreMesh(core_axis_name="c", subcore_axis_name="s")

    @functools.partial(
        pl.kernel,
        out_type=(
            tuple(jax.ShapeDtypeStruct((n, q), jnp.bfloat16) for _ in range(4))
            + (jax.ShapeDtypeStruct((n, 8), jnp.float32),)
        ),
        mesh=mesh,
        scratch_types=[
            pltpu.VMEM((2 * bsz, _IW), jnp.int32),
            pltpu.VMEM((2 * bsz, _IW), jnp.int32),
            pltpu.VMEM((2 * bsz, _IW, q), jnp.bfloat16),
            pltpu.VMEM((_CH, q), jnp.bfloat16),
            pltpu.VMEM((_CH, 8), jnp.float32),
            pltpu.VMEM((_IW, 8), jnp.float32),
            pltpu.VMEM_SHARED((npad, q), jnp.bfloat16),
            pltpu.VMEM_SHARED((npad, 8), jnp.float32),
            pltpu.SemaphoreType.DMA,
            pltpu.SemaphoreType.DMA,
        ],
        compiler_params=pltpu.CompilerParams(use_tc_tiling_on_sc=False),
    )
    def k(h0_hbm, h1_hbm, h2_hbm, h3_hbm, src_hbm, dst_hbm,
          zrow_hbm, zc8_hbm, ones8_hbm,
          a0_hbm, a1_hbm, a2_hbm, a3_hbm, cnt8_hbm,
          src_v, dst_v, rows_v, buf_v, cbuf_v, ones_v, agg_s, cnt_s,
          sem_g, sem_s):
        c = lax.axis_index("c")
        s = lax.axis_index("s")

        pltpu.sync_copy(zc8_hbm, cbuf_v)
        pltpu.sync_copy(ones8_hbm, ones_v)

        def zero_cnt_chunk(r0):
            pltpu.sync_copy(cbuf_v, cnt_s.at[pl.ds(r0, _CH)])

        @pl.when(c == 0)
        def _():
            _chunk_loop(s, n, zero_cnt_chunk)

        def one_pass(table, out_ref, with_cnt):
            pltpu.sync_copy(zrow_hbm, buf_v)

            def zero_chunk(r0):
                pltpu.sync_copy(buf_v, agg_s.at[pl.ds(r0, _CH)])
            _chunk_loop(s, n, zero_chunk)

            plsc.subcore_barrier()

            _edge_pipeline(src_hbm, dst_hbm, table, agg_s,
                           src_v, dst_v, rows_v, sem_g, sem_s,
                           s * rpt, rpt, bsz,
                           cnt_s=cnt_s if with_cnt else None, ones_v=ones_v)

            plsc.subcore_barrier()

            def wb_chunk(r0):
                pltpu.sync_copy(agg_s.at[pl.ds(r0, _CH)], buf_v)
                pltpu.sync_copy(buf_v, out_ref.at[pl.ds(r0, _CH)])
            _chunk_loop(s, n, wb_chunk)

            plsc.subcore_barrier()

        @pl.when(c == 0)
        def _():
            one_pass(h0_hbm, a0_hbm, True)
            one_pass(h1_hbm, a1_hbm, False)

            def wb_cnt_chunk(r0):
                pltpu.sync_copy(cnt_s.at[pl.ds(r0, _CH)], cbuf_v)
                pltpu.sync_copy(cbuf_v, cnt8_hbm.at[pl.ds(r0, _CH)])
            _chunk_loop(s, n, wb_cnt_chunk)

        @pl.when(c == 1)
        def _():
            one_pass(h2_hbm, a2_hbm, False)
            one_pass(h3_hbm, a3_hbm, False)

    return k(*hs, src2d, dst2d, zrow, zc8, ones8)


def _sc_agg_narrow(p, src2d, dst2d, zrow):
    n, w = p.shape
    rows_total = src2d.shape[0]
    rpc = rows_total // _NC
    rpt = rpc // _NS
    bsz = 4
    npad = n + _IW
    mesh = plsc.VectorSubcoreMesh(core_axis_name="c", subcore_axis_name="s")

    @functools.partial(
        pl.kernel,
        out_type=(
            jax.ShapeDtypeStruct((n, w), jnp.bfloat16),
            jax.ShapeDtypeStruct((n, w), jnp.bfloat16),
        ),
        mesh=mesh,
        scratch_types=[
            pltpu.VMEM((2 * bsz, _IW), jnp.int32),
            pltpu.VMEM((2 * bsz, _IW), jnp.int32),
            pltpu.VMEM((2 * bsz, _IW, w), jnp.bfloat16),
            pltpu.VMEM((_CH, w), jnp.bfloat16),
            pltpu.VMEM_SHARED((npad, w), jnp.bfloat16),
            pltpu.SemaphoreType.DMA,
            pltpu.SemaphoreType.DMA,
        ],
        compiler_params=pltpu.CompilerParams(use_tc_tiling_on_sc=False),
    )
    def k(p_hbm, src_hbm, dst_hbm, zrow_hbm, a2a_hbm, a2b_hbm,
          src_v, dst_v, rows_v, buf_v, agg_s, sem_g, sem_s):
        c = lax.axis_index("c")
        s = lax.axis_index("s")

        pltpu.sync_copy(zrow_hbm, buf_v)

        def zero_chunk(r0):
            pltpu.sync_copy(buf_v, agg_s.at[pl.ds(r0, _CH)])
        _chunk_loop(s, n, zero_chunk)

        plsc.subcore_barrier()

        _edge_pipeline(src_hbm, dst_hbm, p_hbm, agg_s,
                       src_v, dst_v, rows_v, sem_g, sem_s,
                       c * rpc + s * rpt, rpt, bsz)

        plsc.subcore_barrier()

        def wb_chunk_a(r0):
            pltpu.sync_copy(agg_s.at[pl.ds(r0, _CH)], buf_v)
            pltpu.sync_copy(buf_v, a2a_hbm.at[pl.ds(r0, _CH)])

        def wb_chunk_b(r0):
            pltpu.sync_copy(agg_s.at[pl.ds(r0, _CH)], buf_v)
            pltpu.sync_copy(buf_v, a2b_hbm.at[pl.ds(r0, _CH)])

        @pl.when(c == 0)
        def _():
            _chunk_loop(s, n, wb_chunk_a)

        @pl.when(c == 1)
        def _():
            _chunk_loop(s, n, wb_chunk_b)

    return k(p, src2d, dst2d, zrow)


def kernel(x, edge_index, proj_W, proj_b, l1_Wl, l1_bl, l1_Wr,
           l2_Wl, l2_bl, l2_Wr):
    n, d = x.shape
    e = edge_index.shape[1]
    h = l1_Wl.shape[0]
    c = l2_Wl.shape[0]
    half = d // 2

    rows_needed = -(-e // _IW)
    rows_total = ((rows_needed + 255) // 256) * 256
    epad = rows_total * _IW
    src = edge_index[0]
    dst = edge_index[1]
    srcp = jnp.concatenate(
        [src, jnp.zeros((epad - e,), jnp.int32)]).reshape(rows_total, _IW)
    pad_dst = n + (jnp.arange(epad - e, dtype=jnp.int32) % _IW)
    dstp = jnp.concatenate([dst, pad_dst]).reshape(rows_total, _IW)

    q = d // 4
    wpT = proj_W.T
    bp = proj_b.reshape(1, d)
    wlT = l1_Wl.T
    wls = [wlT[i * q:(i + 1) * q] for i in range(4)]
    bl = l1_bl.reshape(1, h)
    wrT = l1_Wr.T
    cp = 64
    w2 = jnp.zeros((h, cp), jnp.float32).at[:, :c].set(l2_Wl.T)
    wr2 = jnp.zeros((h, cp), jnp.float32).at[:, :c].set(l2_Wr.T)
    b2 = jnp.zeros((1, cp), jnp.float32).at[:, :c].set(l2_bl.reshape(1, c))

    zrow = jnp.zeros((_CH, q), jnp.bfloat16)
    zc8 = jnp.zeros((_CH, 8), jnp.float32)
    ones8 = jnp.ones((_IW, 8), jnp.float32)
    zp = jnp.zeros((_CH, cp), jnp.bfloat16)

    hs = _tc1(x, wpT, bp)
    a0, a1, a2, a3, cnt8 = _sc_agg_wide(hs, srcp, dstp, zrow, zc8, ones8)
    h1full, p = _tc2([a0, a1, a2, a3], cnt8, x, wls, bl, wrT, w2)
    a2a, a2b = _sc_agg_narrow(p, srcp, dstp, zp)
    out = _tc3(a2a, a2b, cnt8, h1full, wr2, b2, c)
    return out[:, :c]
